# Initial kernel scaffold; baseline (speedup 1.0000x reference)
import jax
import jax.numpy as jnp
from jax.experimental import pallas as pl


def kernel(emb, w, b, y):
    raise NotImplementedError("write your pallas kernel here")


# speedup vs baseline: 8.8821x; 8.8821x over previous
import functools

import jax
import jax.numpy as jnp
from jax.experimental import pallas as pl
from jax.experimental.pallas import tpu as pltpu

N = 4096
D = 1024
C = 128


def _ge2e_kernel(emb_ref, y_ref, w_ref, b_ref, out_ref):
    E = emb_ref[...]
    Esq = E * E
    rn2_col = jnp.sum(Esq, axis=1, keepdims=True)
    ones_row = jnp.ones((1, D), dtype=jnp.float32)
    rn2_row = jax.lax.dot_general(
        ones_row, Esq, (((1,), (1,)), ((), ())),
        preferred_element_type=jnp.float32)

    rn_col = jnp.sqrt(rn2_col)
    e = E / jnp.maximum(rn_col, 1e-12)

    rn_row = jnp.sqrt(rn2_row)
    se_row = rn_row / jnp.maximum(rn_row, 1e-12)
    se2_row = se_row * se_row
    norm_e = jnp.maximum(se_row, 1e-8)

    yv = y_ref[...]
    kio = jax.lax.broadcasted_iota(jnp.int32, (C, N), 0)
    ohb = kio == yv
    oh = ohb.astype(jnp.float32)

    counts = jnp.sum(oh, axis=1, keepdims=True)
    cent = jax.lax.dot_general(
        oh, e, (((1,), (0,)), ((), ())),
        preferred_element_type=jnp.float32)
    csq = jnp.sum(cent * cent, axis=1, keepdims=True)
    GT = jax.lax.dot_general(
        cent, e, (((1,), (1,)), ((), ())),
        preferred_element_type=jnp.float32)

    norm_co = jnp.maximum(jnp.sqrt(csq) / counts, 1e-8)
    S_other = GT / (counts * norm_co * norm_e)

    Gdiag = jnp.sum(oh * GT, axis=0, keepdims=True)
    n_y = jnp.sum(oh * counts, axis=0, keepdims=True)
    csq_y = jnp.sum(oh * csq, axis=0, keepdims=True)

    nm1 = n_y - 1.0
    num_own = (Gdiag - se2_row) / nm1
    own_sq = jnp.maximum(csq_y - 2.0 * Gdiag + se2_row, 0.0)
    norm_own = jnp.maximum(jnp.sqrt(own_sq) / nm1, 1e-8)
    S_own = num_own / (norm_e * norm_own)

    w = w_ref[0]
    b = b_ref[0]
    Sim = jnp.where(ohb, S_own, S_other) * w + b

    m = jnp.max(Sim, axis=1, keepdims=True)
    lse = jnp.log(jnp.sum(jnp.exp(Sim - m), axis=1, keepdims=True)) + m

    contrib = jnp.where(ohb, lse - Sim, 0.0)
    col = jnp.sum(contrib, axis=1, keepdims=True)
    out_ref[...] = jnp.sum(col, axis=0, keepdims=True)


@jax.jit
def _ge2e(emb, w, b, y):
    y2 = y.astype(jnp.int32).reshape(1, N)
    out = pl.pallas_call(
        _ge2e_kernel,
        out_shape=jax.ShapeDtypeStruct((1, 1), jnp.float32),
        in_specs=[
            pl.BlockSpec(memory_space=pltpu.VMEM),
            pl.BlockSpec(memory_space=pltpu.VMEM),
            pl.BlockSpec(memory_space=pltpu.SMEM),
            pl.BlockSpec(memory_space=pltpu.SMEM),
        ],
        out_specs=pl.BlockSpec(memory_space=pltpu.VMEM),
    )(emb, y2, w.reshape(1), b.reshape(1))
    return out[0, 0]


def kernel(emb, w, b, y):
    return _ge2e(emb, w, b, y)


# speedup vs baseline: 8.9765x; 1.0106x over previous
import jax
import jax.numpy as jnp
from jax.experimental import pallas as pl
from jax.experimental.pallas import tpu as pltpu

N = 4096
D = 1024
C = 128
NB = 16
BLK = N // NB


def _ge2e_kernel(emb_hbm, y_ref, w_ref, b_ref, out_ref, e_scr, sem):
    cps = [
        pltpu.make_async_copy(
            emb_hbm.at[pl.ds(i * BLK, BLK), :],
            e_scr.at[pl.ds(i * BLK, BLK), :],
            sem.at[i],
        )
        for i in range(NB)
    ]
    for cp in cps:
        cp.start()

    yv = y_ref[...]
    ones_row = jnp.ones((1, D), dtype=jnp.float32)

    cent = jnp.zeros((C, D), dtype=jnp.float32)
    counts = jnp.zeros((C, 1), dtype=jnp.float32)
    rn2_parts = []
    for i in range(NB):
        cps[i].wait()
        Eb = e_scr[pl.ds(i * BLK, BLK), :]
        Esqb = Eb * Eb
        rn2_b = jax.lax.dot_general(
            ones_row, Esqb, (((1,), (1,)), ((), ())),
            preferred_element_type=jnp.float32)
        rn2_parts.append(rn2_b)
        inv_rb = 1.0 / jnp.maximum(jnp.sqrt(rn2_b), 1e-12)
        yb = yv[:, i * BLK:(i + 1) * BLK]
        kio = jax.lax.broadcasted_iota(jnp.int32, (C, BLK), 0)
        ohb = kio == yb
        ohs = jnp.where(ohb, inv_rb, 0.0)
        counts = counts + jnp.sum(
            ohb.astype(jnp.float32), axis=1, keepdims=True)
        cent = cent + jax.lax.dot_general(
            ohs, Eb, (((1,), (0,)), ((), ())),
            preferred_element_type=jnp.float32)

    rn2_row = jnp.concatenate(rn2_parts, axis=1)
    rn_row = jnp.sqrt(rn2_row)
    inv_r = 1.0 / jnp.maximum(rn_row, 1e-12)
    se_row = rn_row * inv_r
    se2_row = se_row * se_row
    inv_ne = 1.0 / jnp.maximum(se_row, 1e-8)

    csq = jnp.sum(cent * cent, axis=1, keepdims=True)
    inv_n = 1.0 / counts
    norm_co = jnp.maximum(jnp.sqrt(csq) * inv_n, 1e-8)
    A = inv_n / norm_co

    GTr = jax.lax.dot_general(
        cent, e_scr[...], (((1,), (1,)), ((), ())),
        preferred_element_type=jnp.float32)

    w = w_ref[0]
    b = b_ref[0]
    cc = inv_r * inv_ne
    ccw = cc * w

    kio = jax.lax.broadcasted_iota(jnp.int32, (C, N), 0)
    ohb = kio == yv

    Gdiag = jnp.sum(jnp.where(ohb, GTr, 0.0), axis=0, keepdims=True) * inv_r
    n_y = jnp.sum(jnp.where(ohb, counts, 0.0), axis=0, keepdims=True)
    csq_y = jnp.sum(jnp.where(ohb, csq, 0.0), axis=0, keepdims=True)

    inv_nm1 = 1.0 / (n_y - 1.0)
    num_own = (Gdiag - se2_row) * inv_nm1
    own_sq = jnp.maximum(csq_y - 2.0 * Gdiag + se2_row, 0.0)
    norm_own = jnp.maximum(jnp.sqrt(own_sq) * inv_nm1, 1e-8)
    S_own = num_own * inv_ne / norm_own
    SimOwn = S_own * w + b

    Sim = jnp.where(ohb, SimOwn, (GTr * ccw) * A + b)

    m = jnp.max(Sim, axis=1, keepdims=True)
    lse = jnp.log(jnp.sum(jnp.exp(Sim - m), axis=1, keepdims=True)) + m

    t1 = jnp.sum(counts * lse, axis=0, keepdims=True)
    t2 = jnp.sum(SimOwn, axis=1, keepdims=True)
    out_ref[...] = t1 - t2


@jax.jit
def _ge2e(emb, w, b, y):
    y2 = y.astype(jnp.int32).reshape(1, N)
    out = pl.pallas_call(
        _ge2e_kernel,
        out_shape=jax.ShapeDtypeStruct((1, 1), jnp.float32),
        in_specs=[
            pl.BlockSpec(memory_space=pltpu.MemorySpace.HBM),
            pl.BlockSpec(memory_space=pltpu.VMEM),
            pl.BlockSpec(memory_space=pltpu.SMEM),
            pl.BlockSpec(memory_space=pltpu.SMEM),
        ],
        out_specs=pl.BlockSpec(memory_space=pltpu.VMEM),
        scratch_shapes=[
            pltpu.VMEM((N, D), jnp.float32),
            pltpu.SemaphoreType.DMA((NB,)),
        ],
    )(emb, y2, w.reshape(1), b.reshape(1))
    return out[0, 0]


def kernel(emb, w, b, y):
    return _ge2e(emb, w, b, y)


# speedup vs baseline: 10.5062x; 1.1704x over previous
import jax
import jax.numpy as jnp
from jax.experimental import pallas as pl
from jax.experimental.pallas import tpu as pltpu

N = 4096
D = 1024
C = 128
NB = 8
BLK = N // NB


def _ge2e_kernel(emb_hbm, y_ref, w_ref, b_ref, out_ref, e_scr, sem):
    cps = [
        pltpu.make_async_copy(
            emb_hbm.at[pl.ds(i * BLK, BLK), :],
            e_scr.at[pl.ds(i * BLK, BLK), :],
            sem.at[i],
        )
        for i in range(NB)
    ]
    for cp in cps:
        cp.start()

    ones_row = jnp.ones((1, D), dtype=jnp.float32)

    rn2_parts = []
    for i in range(NB):
        cps[i].wait()
        Eb = e_scr[pl.ds(i * BLK, BLK), :]
        rn2_parts.append(jax.lax.dot_general(
            ones_row, Eb * Eb, (((1,), (1,)), ((), ())),
            preferred_element_type=jnp.float32))
    rn2_row = jnp.concatenate(rn2_parts, axis=1)

    rn_row = jnp.sqrt(rn2_row)
    inv_r = 1.0 / jnp.maximum(rn_row, 1e-12)
    se_row = rn_row * inv_r
    se2_row = se_row * se_row
    inv_ne = 1.0 / jnp.maximum(se_row, 1e-8)

    yv = y_ref[...]
    kio = jax.lax.broadcasted_iota(jnp.int32, (C, N), 0)
    ohb = kio == yv
    counts = jnp.sum(jnp.where(ohb, 1.0, 0.0), axis=1, keepdims=True)

    ohs = jnp.where(ohb, inv_r, 0.0)
    cent = jax.lax.dot_general(
        ohs, e_scr[...], (((1,), (0,)), ((), ())),
        preferred_element_type=jnp.float32)

    csq = jnp.sum(cent * cent, axis=1, keepdims=True)
    inv_n = 1.0 / counts
    norm_co = jnp.maximum(jnp.sqrt(csq) * inv_n, 1e-8)
    A = inv_n / norm_co

    GTr = jax.lax.dot_general(
        cent, e_scr[...], (((1,), (1,)), ((), ())),
        preferred_element_type=jnp.float32)

    w = w_ref[0]
    b = b_ref[0]
    ccw = (inv_r * inv_ne) * w

    Gdiag = jnp.sum(jnp.where(ohb, GTr, 0.0), axis=0, keepdims=True) * inv_r
    n_y = jnp.sum(jnp.where(ohb, counts, 0.0), axis=0, keepdims=True)
    csq_y = jnp.sum(jnp.where(ohb, csq, 0.0), axis=0, keepdims=True)

    inv_nm1 = 1.0 / (n_y - 1.0)
    num_own = (Gdiag - se2_row) * inv_nm1
    own_sq = jnp.maximum(csq_y - 2.0 * Gdiag + se2_row, 0.0)
    norm_own = jnp.maximum(jnp.sqrt(own_sq) * inv_nm1, 1e-8)
    S_own = num_own * inv_ne / norm_own
    SimOwn = S_own * w + b

    Sim = jnp.where(ohb, SimOwn, (GTr * ccw) * A + b)

    m = jnp.max(Sim, axis=1, keepdims=True)
    lse = jnp.log(jnp.sum(jnp.exp(Sim - m), axis=1, keepdims=True)) + m

    t1 = jnp.sum(counts * lse, axis=0, keepdims=True)
    t2 = jnp.sum(SimOwn, axis=1, keepdims=True)
    out_ref[...] = t1 - t2


@jax.jit
def _ge2e(emb, w, b, y):
    y2 = y.astype(jnp.int32).reshape(1, N)
    out = pl.pallas_call(
        _ge2e_kernel,
        out_shape=jax.ShapeDtypeStruct((1, 1), jnp.float32),
        in_specs=[
            pl.BlockSpec(memory_space=pltpu.MemorySpace.HBM),
            pl.BlockSpec(memory_space=pltpu.VMEM),
            pl.BlockSpec(memory_space=pltpu.SMEM),
            pl.BlockSpec(memory_space=pltpu.SMEM),
        ],
        out_specs=pl.BlockSpec(memory_space=pltpu.VMEM),
        scratch_shapes=[
            pltpu.VMEM((N, D), jnp.float32),
            pltpu.SemaphoreType.DMA((NB,)),
        ],
    )(emb, y2, w.reshape(1), b.reshape(1))
    return out[0, 0]


def kernel(emb, w, b, y):
    return _ge2e(emb, w, b, y)


# speedup vs baseline: 10.6722x; 1.0158x over previous
import jax
import jax.numpy as jnp
from jax.experimental import pallas as pl
from jax.experimental.pallas import tpu as pltpu

N = 4096
D = 1024
C = 128
NB = 8
BLK = N // NB


def _ge2e_kernel(emb_hbm, y_ref, w_ref, b_ref, out_ref, e_scr, ebf_scr, sem):
    cps = [
        pltpu.make_async_copy(
            emb_hbm.at[pl.ds(i * BLK, BLK), :],
            e_scr.at[pl.ds(i * BLK, BLK), :],
            sem.at[i],
        )
        for i in range(NB)
    ]
    for cp in cps:
        cp.start()

    yv = y_ref[...]
    ones_bf = jnp.ones((1, D), dtype=jnp.bfloat16)

    cent = jnp.zeros((C, D), dtype=jnp.float32)
    rn2_parts = []
    for i in range(NB):
        cps[i].wait()
        Eb = e_scr[pl.ds(i * BLK, BLK), :]
        Esq_bf = (Eb * Eb).astype(jnp.bfloat16)
        rn2_b = jax.lax.dot_general(
            ones_bf, Esq_bf, (((1,), (1,)), ((), ())),
            preferred_element_type=jnp.float32)
        rn2_parts.append(rn2_b)
        Ebf_b = Eb.astype(jnp.bfloat16)
        ebf_scr[pl.ds(i * BLK, BLK), :] = Ebf_b
        inv_rb = 1.0 / jnp.maximum(jnp.sqrt(rn2_b), 1e-12)
        yb = yv[:, i * BLK:(i + 1) * BLK]
        kio = jax.lax.broadcasted_iota(jnp.int32, (C, BLK), 0)
        ohb = kio == yb
        ohs_bf = jnp.where(ohb, inv_rb, 0.0).astype(jnp.bfloat16)
        cent = cent + jax.lax.dot_general(
            ohs_bf, Ebf_b, (((1,), (0,)), ((), ())),
            preferred_element_type=jnp.float32)

    rn2_row = jnp.concatenate(rn2_parts, axis=1)
    rn_row = jnp.sqrt(rn2_row)
    inv_r = 1.0 / jnp.maximum(rn_row, 1e-12)
    se_row = rn_row * inv_r
    se2_row = se_row * se_row
    inv_ne = 1.0 / jnp.maximum(se_row, 1e-8)

    kio = jax.lax.broadcasted_iota(jnp.int32, (C, N), 0)
    ohb = kio == yv
    counts = jnp.sum(jnp.where(ohb, 1.0, 0.0), axis=1, keepdims=True)

    csq = jnp.sum(cent * cent, axis=1, keepdims=True)
    inv_n = 1.0 / counts
    norm_co = jnp.maximum(jnp.sqrt(csq) * inv_n, 1e-8)
    A = inv_n / norm_co

    GTr = jax.lax.dot_general(
        cent.astype(jnp.bfloat16), ebf_scr[...], (((1,), (1,)), ((), ())),
        preferred_element_type=jnp.float32)

    w = w_ref[0]
    b = b_ref[0]
    ccw = (inv_r * inv_ne) * w

    Gdiag = jnp.sum(jnp.where(ohb, GTr, 0.0), axis=0, keepdims=True) * inv_r
    n_y = jnp.sum(jnp.where(ohb, counts, 0.0), axis=0, keepdims=True)
    csq_y = jnp.sum(jnp.where(ohb, csq, 0.0), axis=0, keepdims=True)

    inv_nm1 = 1.0 / (n_y - 1.0)
    num_own = (Gdiag - se2_row) * inv_nm1
    own_sq = jnp.maximum(csq_y - 2.0 * Gdiag + se2_row, 0.0)
    norm_own = jnp.maximum(jnp.sqrt(own_sq) * inv_nm1, 1e-8)
    S_own = num_own * inv_ne / norm_own
    SimOwn = S_own * w + b

    Sim = jnp.where(ohb, SimOwn, (GTr * ccw) * A + b)

    m = jnp.max(Sim, axis=1, keepdims=True)
    lse = jnp.log(jnp.sum(jnp.exp(Sim - m), axis=1, keepdims=True)) + m

    t1 = jnp.sum(counts * lse, axis=0, keepdims=True)
    t2 = jnp.sum(SimOwn, axis=1, keepdims=True)
    out_ref[...] = t1 - t2


@jax.jit
def _ge2e(emb, w, b, y):
    y2 = y.astype(jnp.int32).reshape(1, N)
    out = pl.pallas_call(
        _ge2e_kernel,
        out_shape=jax.ShapeDtypeStruct((1, 1), jnp.float32),
        in_specs=[
            pl.BlockSpec(memory_space=pltpu.MemorySpace.HBM),
            pl.BlockSpec(memory_space=pltpu.VMEM),
            pl.BlockSpec(memory_space=pltpu.SMEM),
            pl.BlockSpec(memory_space=pltpu.SMEM),
        ],
        out_specs=pl.BlockSpec(memory_space=pltpu.VMEM),
        scratch_shapes=[
            pltpu.VMEM((N, D), jnp.float32),
            pltpu.VMEM((N, D), jnp.bfloat16),
            pltpu.SemaphoreType.DMA((NB,)),
        ],
    )(emb, y2, w.reshape(1), b.reshape(1))
    return out[0, 0]


def kernel(emb, w, b, y):
    return _ge2e(emb, w, b, y)


# speedup vs baseline: 10.6732x; 1.0001x over previous
import jax
import jax.numpy as jnp
from jax.experimental import pallas as pl
from jax.experimental.pallas import tpu as pltpu

N = 4096
D = 1024
C = 128
NB = 4
BLK = N // NB


def _ge2e_kernel(emb_hbm, y_ref, w_ref, b_ref, out_ref, e_scr, ebf_scr, sem):
    cps = [
        pltpu.make_async_copy(
            emb_hbm.at[pl.ds(i * BLK, BLK), :],
            e_scr.at[pl.ds(i * BLK, BLK), :],
            sem.at[i],
        )
        for i in range(NB)
    ]
    for cp in cps:
        cp.start()

    yv = y_ref[...]
    ones_bf = jnp.ones((1, D), dtype=jnp.bfloat16)

    cent = jnp.zeros((C, D), dtype=jnp.float32)
    rn2_parts = []
    for i in range(NB):
        cps[i].wait()
        Eb = e_scr[pl.ds(i * BLK, BLK), :]
        Esq_bf = (Eb * Eb).astype(jnp.bfloat16)
        rn2_b = jax.lax.dot_general(
            ones_bf, Esq_bf, (((1,), (1,)), ((), ())),
            preferred_element_type=jnp.float32)
        rn2_parts.append(rn2_b)
        Ebf_b = Eb.astype(jnp.bfloat16)
        ebf_scr[pl.ds(i * BLK, BLK), :] = Ebf_b
        inv_rb = 1.0 / jnp.maximum(jnp.sqrt(rn2_b), 1e-12)
        yb = yv[:, i * BLK:(i + 1) * BLK]
        kio = jax.lax.broadcasted_iota(jnp.int32, (C, BLK), 0)
        ohb = kio == yb
        ohs_bf = jnp.where(ohb, inv_rb, 0.0).astype(jnp.bfloat16)
        cent = cent + jax.lax.dot_general(
            ohs_bf, Ebf_b, (((1,), (0,)), ((), ())),
            preferred_element_type=jnp.float32)

    rn2_row = jnp.concatenate(rn2_parts, axis=1)
    rn_row = jnp.sqrt(rn2_row)
    inv_r = 1.0 / jnp.maximum(rn_row, 1e-12)
    se_row = rn_row * inv_r
    se2_row = se_row * se_row
    inv_ne = 1.0 / jnp.maximum(se_row, 1e-8)

    kio = jax.lax.broadcasted_iota(jnp.int32, (C, N), 0)
    ohb = kio == yv
    counts = jnp.sum(jnp.where(ohb, 1.0, 0.0), axis=1, keepdims=True)

    csq = jnp.sum(cent * cent, axis=1, keepdims=True)
    inv_n = 1.0 / counts
    norm_co = jnp.maximum(jnp.sqrt(csq) * inv_n, 1e-8)
    A = inv_n / norm_co

    GTr = jax.lax.dot_general(
        cent.astype(jnp.bfloat16), ebf_scr[...], (((1,), (1,)), ((), ())),
        preferred_element_type=jnp.float32)

    w = w_ref[0]
    b = b_ref[0]
    ccw = (inv_r * inv_ne) * w

    Gdiag = jnp.sum(jnp.where(ohb, GTr, 0.0), axis=0, keepdims=True) * inv_r
    n_y = jnp.sum(jnp.where(ohb, counts, 0.0), axis=0, keepdims=True)
    csq_y = jnp.sum(jnp.where(ohb, csq, 0.0), axis=0, keepdims=True)

    inv_nm1 = 1.0 / (n_y - 1.0)
    num_own = (Gdiag - se2_row) * inv_nm1
    own_sq = jnp.maximum(csq_y - 2.0 * Gdiag + se2_row, 0.0)
    norm_own = jnp.maximum(jnp.sqrt(own_sq) * inv_nm1, 1e-8)
    S_own = num_own * inv_ne / norm_own
    SimOwn = S_own * w + b

    Sim = jnp.where(ohb, SimOwn, (GTr * ccw) * A + b)

    m = jnp.max(Sim, axis=1, keepdims=True)
    lse = jnp.log(jnp.sum(jnp.exp(Sim - m), axis=1, keepdims=True)) + m

    t1 = jnp.sum(counts * lse, axis=0, keepdims=True)
    t2 = jnp.sum(SimOwn, axis=1, keepdims=True)
    out_ref[...] = t1 - t2


@jax.jit
def _ge2e(emb, w, b, y):
    y2 = y.astype(jnp.int32).reshape(1, N)
    out = pl.pallas_call(
        _ge2e_kernel,
        out_shape=jax.ShapeDtypeStruct((1, 1), jnp.float32),
        in_specs=[
            pl.BlockSpec(memory_space=pltpu.MemorySpace.HBM),
            pl.BlockSpec(memory_space=pltpu.VMEM),
            pl.BlockSpec(memory_space=pltpu.SMEM),
            pl.BlockSpec(memory_space=pltpu.SMEM),
        ],
        out_specs=pl.BlockSpec(memory_space=pltpu.VMEM),
        scratch_shapes=[
            pltpu.VMEM((N, D), jnp.float32),
            pltpu.VMEM((N, D), jnp.bfloat16),
            pltpu.SemaphoreType.DMA((NB,)),
        ],
    )(emb, y2, w.reshape(1), b.reshape(1))
    return out[0, 0]


def kernel(emb, w, b, y):
    return _ge2e(emb, w, b, y)
